# baseline (speedup 1.0000x reference)
import numpy as np
import jax
import jax.numpy as jnp
from jax.experimental import pallas as pl

N_CELLS = 256
N_GENES = 2000
N_EMB = 100
N_BIAS = 128

ROW = 3200
SEGS_PER_ROW = ROW // N_EMB
N_ROWS = N_CELLS * N_GENES * N_EMB // ROW
R_BLK = 1000

_MASK = np.zeros((ROW, SEGS_PER_ROW), dtype=np.float32)
for _e in range(ROW):
    _MASK[_e, _e // N_EMB] = 1.0


def _mean_kernel(x_ref, m_ref, s_ref):
    x = x_ref[...]
    m = m_ref[...]
    s = jax.lax.dot_general(
        x, m, (((1,), (0,)), ((), ())),
        preferred_element_type=jnp.float32,
    )
    s_ref[...] = s * (1.0 / N_EMB)


def _bias_kernel(s_ref, gix_ref, bias_ref, out_ref):
    s = s_ref[...]
    gix = gix_ref[...]
    bias = bias_ref[...]
    col = jax.lax.broadcasted_iota(jnp.int32, (N_GENES, N_BIAS), 1)
    onehot = gix[0][:, None] == col
    brow = jnp.sum(jnp.where(onehot, bias, 0.0), axis=1)
    out_ref[...] = s + brow[None, :]


@jax.jit
def kernel(cell_gene_embedding, gene_ix, bias1):
    x2 = cell_gene_embedding.reshape(N_ROWS, ROW)
    mask = jnp.asarray(_MASK)
    seg_sums = pl.pallas_call(
        _mean_kernel,
        grid=(N_ROWS // R_BLK,),
        in_specs=[
            pl.BlockSpec((R_BLK, ROW), lambda i: (i, 0)),
            pl.BlockSpec((ROW, SEGS_PER_ROW), lambda i: (0, 0)),
        ],
        out_specs=pl.BlockSpec((R_BLK, SEGS_PER_ROW), lambda i: (i, 0)),
        out_shape=jax.ShapeDtypeStruct((N_ROWS, SEGS_PER_ROW), jnp.float32),
    )(x2, mask)
    means = seg_sums.reshape(N_CELLS, N_GENES)
    gix2 = gene_ix.astype(jnp.int32).reshape(1, N_GENES)
    bias2 = bias1.reshape(1, N_BIAS)
    return pl.pallas_call(
        _bias_kernel,
        in_specs=[
            pl.BlockSpec((N_CELLS, N_GENES), lambda: (0, 0)),
            pl.BlockSpec((1, N_GENES), lambda: (0, 0)),
            pl.BlockSpec((1, N_BIAS), lambda: (0, 0)),
        ],
        out_specs=pl.BlockSpec((N_CELLS, N_GENES), lambda: (0, 0)),
        out_shape=jax.ShapeDtypeStruct((N_CELLS, N_GENES), jnp.float32),
    )(means, gix2, bias2)


# speedup vs baseline: 1.5820x; 1.5820x over previous
import jax
import jax.numpy as jnp
from jax.experimental import pallas as pl

C_BLK = 8
N_GENES = 2000
N_EMB = 100
N_BIAS = 128


def _mean_bias_kernel(emb_ref, gix_ref, bias_ref, out_ref):
    x = emb_ref[...]
    s = jnp.sum(x, axis=-1) * (1.0 / N_EMB)
    gix = gix_ref[...]
    bias = bias_ref[...]
    col = jax.lax.broadcasted_iota(jnp.int32, (N_GENES, N_BIAS), 1)
    onehot = gix[0][:, None] == col
    bvals = jnp.sum(jnp.where(onehot, bias, 0.0), axis=1)
    out_ref[...] = s + bvals[None, :]


@jax.jit
def kernel(cell_gene_embedding, gene_ix, bias1):
    n_cells = cell_gene_embedding.shape[0]
    gix2 = gene_ix.astype(jnp.int32).reshape(1, N_GENES)
    bias2 = bias1.reshape(1, N_BIAS)
    grid = (n_cells // C_BLK,)
    return pl.pallas_call(
        _mean_bias_kernel,
        grid=grid,
        in_specs=[
            pl.BlockSpec((C_BLK, N_GENES, N_EMB), lambda i: (i, 0, 0)),
            pl.BlockSpec((1, N_GENES), lambda i: (0, 0)),
            pl.BlockSpec((1, N_BIAS), lambda i: (0, 0)),
        ],
        out_specs=pl.BlockSpec((C_BLK, N_GENES), lambda i: (i, 0)),
        out_shape=jax.ShapeDtypeStruct((n_cells, N_GENES), jnp.float32),
    )(cell_gene_embedding, gix2, bias2)


# speedup vs baseline: 1.7570x; 1.1106x over previous
import jax
import jax.numpy as jnp
from jax.experimental import pallas as pl

N_STREAMS = 4
C_SUB = 4
C_BLK = N_STREAMS * C_SUB
N_GENES = 2000
N_EMB = 100
N_BIAS = 128


def _mean_bias_kernel(x0, x1, x2, x3, gix_ref, bias_ref, out_ref):
    gix = gix_ref[...]
    bias = bias_ref[...]
    col = jax.lax.broadcasted_iota(jnp.int32, (N_GENES, N_BIAS), 1)
    onehot = gix[0][:, None] == col
    brow = jnp.sum(jnp.where(onehot, bias, 0.0), axis=1)[None, :]
    for q, xr in enumerate((x0, x1, x2, x3)):
        s = jnp.sum(xr[...], axis=-1) * (1.0 / N_EMB)
        out_ref[q * C_SUB:(q + 1) * C_SUB, :] = s + brow


@jax.jit
def kernel(cell_gene_embedding, gene_ix, bias1):
    n_cells = cell_gene_embedding.shape[0]
    gix2 = gene_ix.astype(jnp.int32).reshape(1, N_GENES)
    bias2 = bias1.reshape(1, N_BIAS)
    grid = (n_cells // C_BLK,)

    def make_spec(q):
        return pl.BlockSpec(
            (C_SUB, N_GENES, N_EMB),
            lambda i, q=q: (i * N_STREAMS + q, 0, 0),
        )

    return pl.pallas_call(
        _mean_bias_kernel,
        grid=grid,
        in_specs=[make_spec(q) for q in range(N_STREAMS)]
        + [
            pl.BlockSpec((1, N_GENES), lambda i: (0, 0)),
            pl.BlockSpec((1, N_BIAS), lambda i: (0, 0)),
        ],
        out_specs=pl.BlockSpec((C_BLK, N_GENES), lambda i: (i, 0)),
        out_shape=jax.ShapeDtypeStruct((n_cells, N_GENES), jnp.float32),
    )(
        cell_gene_embedding,
        cell_gene_embedding,
        cell_gene_embedding,
        cell_gene_embedding,
        gix2,
        bias2,
    )
